# baseline (speedup 1.0000x reference)
import functools

import jax
import jax.numpy as jnp
from jax import lax
from jax.experimental import pallas as pl
from jax.experimental.pallas import tpu as pltpu
from jax.experimental.pallas import tpu_sc as plsc


BATCH_BLOCK = 128


def _add_kernel(x_ref, pe_ref, o_ref):
    o_ref[...] = x_ref[...] + pe_ref[...]


def _kernel_tc(x, pos_emb):
    B, L, D = x.shape
    pe = pos_emb[:L]
    grid = (B // BATCH_BLOCK,)
    return pl.pallas_call(
        _add_kernel,
        grid=grid,
        in_specs=[
            pl.BlockSpec((BATCH_BLOCK, L, D), lambda i: (i, 0, 0)),
            pl.BlockSpec((L, D), lambda i: (0, 0)),
        ],
        out_specs=pl.BlockSpec((BATCH_BLOCK, L, D), lambda i: (i, 0, 0)),
        out_shape=jax.ShapeDtypeStruct((B, L, D), x.dtype),
    )(x, pe)


_NC = 2
_NS = 16
_NW = _NC * _NS
_LANES = 16


def _make_sc(B, L, D):
    chunk = L * D
    bpw = B // _NW
    n = B * L * D

    mesh = plsc.VectorSubcoreMesh(core_axis_name="c", subcore_axis_name="s")

    @functools.partial(
        pl.kernel,
        mesh=mesh,
        out_type=jax.ShapeDtypeStruct((n,), jnp.float32),
        scratch_types=[
            pltpu.VMEM((chunk,), jnp.float32),
            pltpu.VMEM((chunk,), jnp.float32),
        ],
    )
    def sc_add(x_hbm, pe_hbm, out_hbm, pe_v, buf_v):
        wid = lax.axis_index("s") * _NC + lax.axis_index("c")
        pltpu.sync_copy(pe_hbm, pe_v)

        def inner(i, carry):
            s = pl.multiple_of(i * _LANES, _LANES)
            buf_v[pl.ds(s, _LANES)] = buf_v[pl.ds(s, _LANES)] + pe_v[pl.ds(s, _LANES)]
            return carry

        def body(b, carry):
            base = pl.multiple_of((wid * bpw + b) * chunk, 8)
            pltpu.sync_copy(x_hbm.at[pl.ds(base, chunk)], buf_v)
            lax.fori_loop(0, chunk // _LANES, inner, 0)
            pltpu.sync_copy(buf_v, out_hbm.at[pl.ds(base, chunk)])
            return carry

        lax.fori_loop(0, bpw, body, 0)

    return sc_add


def _kernel_sc(x, pos_emb):
    B, L, D = x.shape
    pe = pos_emb[:L]
    out = _make_sc(B, L, D)(x.reshape(-1), pe.reshape(-1))
    return out.reshape(B, L, D)


def kernel(x, pos_emb):
    return _kernel_sc(x, pos_emb)


# speedup vs baseline: 3.4341x; 3.4341x over previous
import functools

import jax
import jax.numpy as jnp
from jax import lax
from jax.experimental import pallas as pl
from jax.experimental.pallas import tpu as pltpu
from jax.experimental.pallas import tpu_sc as plsc


BATCH_BLOCK = 128


def _add_kernel(x_ref, pe_ref, o_ref):
    o_ref[...] = x_ref[...] + pe_ref[...]


def _kernel_tc(x, pos_emb):
    B, L, D = x.shape
    pe = pos_emb[:L]
    grid = (B // BATCH_BLOCK,)
    return pl.pallas_call(
        _add_kernel,
        grid=grid,
        in_specs=[
            pl.BlockSpec((BATCH_BLOCK, L, D), lambda i: (i, 0, 0)),
            pl.BlockSpec((L, D), lambda i: (0, 0)),
        ],
        out_specs=pl.BlockSpec((BATCH_BLOCK, L, D), lambda i: (i, 0, 0)),
        out_shape=jax.ShapeDtypeStruct((B, L, D), x.dtype),
    )(x, pe)


_NC = 2
_NS = 16
_NW = _NC * _NS
_LANES = 16


def _make_sc(B, L, D):
    chunk = L * D
    bpw = B // _NW
    n = B * L * D

    mesh = plsc.VectorSubcoreMesh(core_axis_name="c", subcore_axis_name="s")

    @functools.partial(
        pl.kernel,
        mesh=mesh,
        out_type=jax.ShapeDtypeStruct((n,), jnp.float32),
        scratch_types=[
            pltpu.VMEM((chunk,), jnp.float32),
            pltpu.VMEM((chunk,), jnp.float32),
            pltpu.VMEM((chunk,), jnp.float32),
            pltpu.SemaphoreType.DMA,
            pltpu.SemaphoreType.DMA,
            pltpu.SemaphoreType.DMA,
            pltpu.SemaphoreType.DMA,
        ],
    )
    def sc_add(x_hbm, pe_hbm, out_hbm, pe_v, a_v, b_v, la, lb, sa, sb):
        wid = lax.axis_index("s") * _NC + lax.axis_index("c")
        pltpu.sync_copy(pe_hbm, pe_v)
        row0 = wid * bpw

        def compute(buf):
            @plsc.parallel_loop(0, chunk // _LANES, 1, unroll=16)
            def _(i):
                s = pl.multiple_of(i * _LANES, _LANES)
                buf[pl.ds(s, _LANES)] = buf[pl.ds(s, _LANES)] + pe_v[pl.ds(s, _LANES)]

        def src(b):
            base = pl.multiple_of((row0 + b) * chunk, 8)
            return x_hbm.at[pl.ds(base, chunk)]

        def dst(b):
            base = pl.multiple_of((row0 + b) * chunk, 8)
            return out_hbm.at[pl.ds(base, chunk)]

        pltpu.async_copy(src(0), a_v, la)

        def body(p, carry):
            b0 = p * 2
            pltpu.make_async_copy(src(b0), a_v, la).wait()

            @pl.when(p > 0)
            def _():
                pltpu.make_async_copy(b_v, dst(b0), sb).wait()

            pltpu.async_copy(src(b0 + 1), b_v, lb)
            compute(a_v)
            pltpu.async_copy(a_v, dst(b0), sa)
            pltpu.make_async_copy(src(b0 + 1), b_v, lb).wait()
            compute(b_v)
            pltpu.make_async_copy(a_v, dst(b0), sa).wait()

            @pl.when(p < bpw // 2 - 1)
            def _():
                pltpu.async_copy(src(b0 + 2), a_v, la)

            pltpu.async_copy(b_v, dst(b0 + 1), sb)
            return carry

        lax.fori_loop(0, bpw // 2, body, 0)
        pltpu.make_async_copy(b_v, dst(bpw - 1), sb).wait()

    return sc_add


def _kernel_sc(x, pos_emb):
    B, L, D = x.shape
    pe = pos_emb[:L]
    out = _make_sc(B, L, D)(x.reshape(-1), pe.reshape(-1))
    return out.reshape(B, L, D)


def kernel(x, pos_emb):
    return _kernel_sc(x, pos_emb)


# speedup vs baseline: 5.1061x; 1.4869x over previous
import functools

import jax
import jax.numpy as jnp
from jax import lax
from jax.experimental import pallas as pl
from jax.experimental.pallas import tpu as pltpu
from jax.experimental.pallas import tpu_sc as plsc


BATCH_BLOCK = 128


def _add_kernel(x_ref, pe_ref, o_ref):
    o_ref[...] = x_ref[...] + pe_ref[...]


def _kernel_tc(x, pos_emb):
    B, L, D = x.shape
    pe = pos_emb[:L]
    grid = (B // BATCH_BLOCK,)
    return pl.pallas_call(
        _add_kernel,
        grid=grid,
        in_specs=[
            pl.BlockSpec((BATCH_BLOCK, L, D), lambda i: (i, 0, 0)),
            pl.BlockSpec((L, D), lambda i: (0, 0)),
        ],
        out_specs=pl.BlockSpec((BATCH_BLOCK, L, D), lambda i: (i, 0, 0)),
        out_shape=jax.ShapeDtypeStruct((B, L, D), x.dtype),
        compiler_params=pltpu.CompilerParams(
            vmem_limit_bytes=60 * 1024 * 1024,
        ),
    )(x, pe)


_NC = 2
_NS = 16
_NW = _NC * _NS
_LANES = 16


def _make_sc(B, L, D):
    chunk = L * D
    bpw = B // _NW
    n = B * L * D

    mesh = plsc.VectorSubcoreMesh(core_axis_name="c", subcore_axis_name="s")

    @functools.partial(
        pl.kernel,
        mesh=mesh,
        out_type=jax.ShapeDtypeStruct((n,), jnp.float32),
        scratch_types=[
            pltpu.VMEM((chunk,), jnp.float32),
            pltpu.VMEM((chunk,), jnp.float32),
            pltpu.VMEM((chunk,), jnp.float32),
            pltpu.SemaphoreType.DMA,
            pltpu.SemaphoreType.DMA,
            pltpu.SemaphoreType.DMA,
            pltpu.SemaphoreType.DMA,
        ],
    )
    def sc_add(x_hbm, pe_hbm, out_hbm, pe_v, a_v, b_v, la, lb, sa, sb):
        wid = lax.axis_index("s") * _NC + lax.axis_index("c")
        pltpu.sync_copy(pe_hbm, pe_v)
        row0 = wid * bpw

        def compute(buf):
            @plsc.parallel_loop(0, chunk // _LANES, 1, unroll=16)
            def _(i):
                s = pl.multiple_of(i * _LANES, _LANES)
                buf[pl.ds(s, _LANES)] = buf[pl.ds(s, _LANES)] + pe_v[pl.ds(s, _LANES)]

        def src(b):
            base = pl.multiple_of((row0 + b) * chunk, 8)
            return x_hbm.at[pl.ds(base, chunk)]

        def dst(b):
            base = pl.multiple_of((row0 + b) * chunk, 8)
            return out_hbm.at[pl.ds(base, chunk)]

        pltpu.async_copy(src(0), a_v, la)

        def body(p, carry):
            b0 = p * 2
            pltpu.make_async_copy(src(b0), a_v, la).wait()

            @pl.when(p > 0)
            def _():
                pltpu.make_async_copy(b_v, dst(b0), sb).wait()

            pltpu.async_copy(src(b0 + 1), b_v, lb)
            compute(a_v)
            pltpu.async_copy(a_v, dst(b0), sa)
            pltpu.make_async_copy(src(b0 + 1), b_v, lb).wait()
            compute(b_v)
            pltpu.make_async_copy(a_v, dst(b0), sa).wait()

            @pl.when(p < bpw // 2 - 1)
            def _():
                pltpu.async_copy(src(b0 + 2), a_v, la)

            pltpu.async_copy(b_v, dst(b0 + 1), sb)
            return carry

        lax.fori_loop(0, bpw // 2, body, 0)
        pltpu.make_async_copy(b_v, dst(bpw - 1), sb).wait()

    return sc_add


def _kernel_sc(x, pos_emb):
    B, L, D = x.shape
    pe = pos_emb[:L]
    out = _make_sc(B, L, D)(x.reshape(-1), pe.reshape(-1))
    return out.reshape(B, L, D)


def kernel(x, pos_emb):
    return _kernel_tc(x, pos_emb)
